# baseline (speedup 1.0000x reference)
import jax
import jax.numpy as jnp
from jax import lax
from jax.experimental import pallas as pl
from jax.experimental.pallas import tpu as pltpu
from jax.experimental.pallas import tpu_sc as plsc

N = 10000
D = 128
R = 3
HD = D // 2
NS = 16
CH = 128
NCH = 158
EP = NS * CH * NCH
NCH2 = EP // (32 * CH)
AROWS = 30016
STRIPE = AROWS // NS
CNTN = 30016
CS = 1872


def _sc_agg():
  mesh = plsc.VectorSubcoreMesh(core_axis_name="c", subcore_axis_name="s",
                                num_cores=2, num_subcores=NS)
  out_type = (jax.ShapeDtypeStruct((AROWS, D), jnp.float32),)
  scratch = (
      pltpu.VMEM((CH,), jnp.int32),
      pltpu.VMEM((CH,), jnp.int32),
      pltpu.VMEM((CH,), jnp.int32),
      pltpu.VMEM((CH,), jnp.int32),
      pltpu.VMEM((CH,), jnp.int32),
      pltpu.VMEM((CH, HD), jnp.float32),
      pltpu.VMEM_SHARED((AROWS, HD), jnp.float32),
      pltpu.SemaphoreType.DMA,
  )

  def body(h2, src, dst, typ, sums, sbuf, dbuf, tbuf, idxg, idxs, rows, acc,
           sem):
    cid = lax.axis_index("c")
    sid = lax.axis_index("s")
    zero16 = jnp.zeros((16,), jnp.float32)

    def zrow_loop(i, c):
      def zcol_loop(j, c2):
        rows[i, pl.ds(j * 16, 16)] = zero16
        return c2
      return lax.fori_loop(0, HD // 16, zcol_loop, c)

    lax.fori_loop(0, CH, zrow_loop, 0)
    for k in range(STRIPE // CH):
      pltpu.sync_copy(rows, acc.at[pl.ds(sid * STRIPE + k * CH, CH), :])
    rem = STRIPE % CH
    if rem:
      pltpu.sync_copy(rows.at[pl.ds(0, rem), :],
                      acc.at[pl.ds(sid * STRIPE + (STRIPE // CH) * CH, rem), :])

    plsc.subcore_barrier()

    ebase = sid * (NCH * CH)

    def chunk(i, carry):
      off = ebase + i * CH
      pltpu.sync_copy(src.at[pl.ds(off, CH)], sbuf)
      pltpu.sync_copy(dst.at[pl.ds(off, CH)], dbuf)
      pltpu.sync_copy(typ.at[pl.ds(off, CH)], tbuf)

      def sub(j, c):
        sl = pl.ds(j * 16, 16)
        idxg[sl] = sbuf[sl] * 2 + cid
        idxs[sl] = tbuf[sl] * N + dbuf[sl]
        return c

      lax.fori_loop(0, CH // 16, sub, 0)
      pltpu.async_copy(h2.at[idxg], rows, sem).wait()
      pltpu.sync_copy(rows, acc.at[idxs], add=True)
      return carry

    lax.fori_loop(0, NCH, chunk, 0)
    plsc.subcore_barrier()

    pltpu.sync_copy(acc.at[pl.ds(sid * STRIPE, STRIPE), :],
                    sums.at[pl.ds(sid * STRIPE, STRIPE), pl.ds(cid * HD, HD)])

  return pl.kernel(body, out_type=out_type, mesh=mesh,
                   scratch_types=scratch,
                   compiler_params=pltpu.CompilerParams(
                       use_tc_tiling_on_sc=False))


def _sc_cnt():
  mesh = plsc.VectorSubcoreMesh(core_axis_name="c", subcore_axis_name="s",
                                num_cores=2, num_subcores=NS)
  out_type = (jax.ShapeDtypeStruct((CNTN,), jnp.float32),
              jax.ShapeDtypeStruct((CNTN,), jnp.float32))
  scratch = (
      pltpu.VMEM((CH,), jnp.int32),
      pltpu.VMEM((CH,), jnp.int32),
      pltpu.VMEM((CH,), jnp.int32),
      pltpu.VMEM((CH,), jnp.float32),
      pltpu.VMEM_SHARED((CNTN,), jnp.float32),
  )

  def body(dst, typ, zcnt, ones_h, cnt0_out, cnt1_out,
           dbuf, tbuf, idxs, ones_v, acccnt):
    cid = lax.axis_index("c")
    sid = lax.axis_index("s")

    pltpu.sync_copy(ones_h, ones_v)

    @pl.when(sid == 0)
    def _():
      pltpu.sync_copy(zcnt, acccnt)

    plsc.subcore_barrier()

    ebase = (cid * NS + sid) * (NCH2 * CH)

    def chunk(i, carry):
      off = ebase + i * CH
      pltpu.sync_copy(dst.at[pl.ds(off, CH)], dbuf)
      pltpu.sync_copy(typ.at[pl.ds(off, CH)], tbuf)

      def sub(j, c):
        sl = pl.ds(j * 16, 16)
        idxs[sl] = tbuf[sl] * N + dbuf[sl]
        return c

      lax.fori_loop(0, CH // 16, sub, 0)
      pltpu.sync_copy(ones_v, acccnt.at[idxs], add=True)
      return carry

    lax.fori_loop(0, NCH2, chunk, 0)
    plsc.subcore_barrier()

    def publish(out_ref):
      pltpu.sync_copy(acccnt.at[pl.ds(sid * CS, CS)],
                      out_ref.at[pl.ds(sid * CS, CS)])

      @pl.when(sid == 0)
      def _():
        pltpu.sync_copy(acccnt.at[pl.ds(NS * CS, CNTN - NS * CS)],
                        out_ref.at[pl.ds(NS * CS, CNTN - NS * CS)])

    @pl.when(cid == 0)
    def _():
      publish(cnt0_out)

    @pl.when(cid == 1)
    def _():
      publish(cnt1_out)

  return pl.kernel(body, out_type=out_type, mesh=mesh,
                   scratch_types=scratch,
                   compiler_params=pltpu.CompilerParams(
                       use_tc_tiling_on_sc=False))


def _ln(h, g, b, eps=1e-5):
  mu = jnp.mean(h, axis=-1, keepdims=True)
  d = h - mu
  var = jnp.mean(d * d, axis=-1, keepdims=True)
  return d * lax.rsqrt(var + eps) * g + b


BR = 400


def _tc_layer(leaky, double_ln):
  def body(*args):
    if double_ln:
      (x_ref, s0_ref, s1_ref, s2_ref, ca_ref, cb_ref,
       w_ref, root_ref, b_ref, g_ref, be_ref, g2_ref, be2_ref, o_ref) = args
    else:
      (x_ref, s0_ref, s1_ref, s2_ref, ca_ref, cb_ref,
       w_ref, root_ref, b_ref, g_ref, be_ref, o_ref) = args

    x = x_ref[...]
    acc = jnp.dot(x, root_ref[...], precision=lax.Precision.HIGHEST,
                  preferred_element_type=jnp.float32) + b_ref[...][None, :]
    invs = 1.0 / jnp.maximum(ca_ref[...] + cb_ref[...], 1.0)
    for r, s_ref in enumerate((s0_ref, s1_ref, s2_ref)):
      mean_r = s_ref[...] * invs[:, r][:, None]
      acc = acc + jnp.dot(mean_r, w_ref[r], precision=lax.Precision.HIGHEST,
                          preferred_element_type=jnp.float32)
    h = _ln(acc, g_ref[...], be_ref[...])
    if leaky:
      h = jnp.where(h > 0, h, 0.2 * h)
    if double_ln:
      h = _ln(h, g2_ref[...], be2_ref[...])
    o_ref[...] = h

  grid = (N // BR,)
  nblk = N // BR
  vec = pl.BlockSpec((D,), lambda i: (0,))
  in_specs = [
      pl.BlockSpec((BR, D), lambda i: (i, 0)),
      pl.BlockSpec((BR, D), lambda i: (i, 0)),
      pl.BlockSpec((BR, D), lambda i: (i + nblk, 0)),
      pl.BlockSpec((BR, D), lambda i: (i + 2 * nblk, 0)),
      pl.BlockSpec((BR, R), lambda i: (i, 0)),
      pl.BlockSpec((BR, R), lambda i: (i, 0)),
      pl.BlockSpec((R, D, D), lambda i: (0, 0, 0)),
      pl.BlockSpec((D, D), lambda i: (0, 0)),
      vec, vec, vec,
  ]
  if double_ln:
    in_specs += [vec, vec]
  return pl.pallas_call(
      body,
      grid=grid,
      in_specs=in_specs,
      out_specs=pl.BlockSpec((BR, D), lambda i: (i, 0)),
      out_shape=jax.ShapeDtypeStruct((N, D), jnp.float32),
  )


def kernel(x, edge_index, edge_type, w0, root0, b0, g0, be0,
           w1, root1, b1, g1, be1, gon, bon):
  E = edge_index.shape[1]
  pad = EP - E
  src = jnp.concatenate(
      [edge_index[0].astype(jnp.int32), jnp.zeros((pad,), jnp.int32)])
  dst = jnp.concatenate(
      [edge_index[1].astype(jnp.int32), jnp.zeros((pad,), jnp.int32)])
  typ = jnp.concatenate(
      [edge_type.astype(jnp.int32), jnp.full((pad,), R, jnp.int32)])

  agg = _sc_agg()
  cntk = _sc_cnt()
  tc0 = _tc_layer(leaky=True, double_ln=False)
  tc1 = _tc_layer(leaky=False, double_ln=True)

  c0, c1 = cntk(dst, typ, jnp.zeros((CNTN,), jnp.float32),
                jnp.ones((CH,), jnp.float32))
  c0t = c0[:R * N].reshape(R, N).T
  c1t = c1[:R * N].reshape(R, N).T
  (sums0,) = agg(x.reshape(2 * N, HD), src, dst, typ)

  h1 = tc0(x, sums0, sums0, sums0, c0t, c1t, w0, root0, b0, g0, be0)

  (sums1,) = agg(h1.reshape(2 * N, HD), src, dst, typ)

  return tc1(h1, sums1, sums1, sums1, c0t, c1t, w1, root1, b1, g1, be1,
             gon, bon)


# speedup vs baseline: 1.0218x; 1.0218x over previous
import jax
import jax.numpy as jnp
from jax import lax
from jax.experimental import pallas as pl
from jax.experimental.pallas import tpu as pltpu
from jax.experimental.pallas import tpu_sc as plsc

N = 10000
D = 128
R = 3
HD = D // 2
NS = 16
CH = 128
NCH = 157
EP = NS * CH * NCH
AROWS = 30016
STRIPE = AROWS // NS
CNTN = 30016
CS = 1872


def _sc_agg():
  mesh = plsc.VectorSubcoreMesh(core_axis_name="c", subcore_axis_name="s",
                                num_cores=2, num_subcores=NS)
  out_type = (jax.ShapeDtypeStruct((AROWS, D), jnp.float32),)
  scratch = (
      pltpu.VMEM((CH,), jnp.int32),
      pltpu.VMEM((CH,), jnp.int32),
      pltpu.VMEM((CH,), jnp.int32),
      pltpu.VMEM((CH,), jnp.int32),
      pltpu.VMEM((CH,), jnp.int32),
      pltpu.VMEM((CH, HD), jnp.float32),
      pltpu.VMEM_SHARED((AROWS, HD), jnp.float32),
      pltpu.SemaphoreType.DMA,
  )

  def body(h2, src, dst, typ, sums, sbuf, dbuf, tbuf, idxg, idxs, rows, acc,
           sem):
    cid = lax.axis_index("c")
    sid = lax.axis_index("s")
    zero16 = jnp.zeros((16,), jnp.float32)

    def zrow_loop(i, c):
      def zcol_loop(j, c2):
        rows[i, pl.ds(j * 16, 16)] = zero16
        return c2
      return lax.fori_loop(0, HD // 16, zcol_loop, c)

    lax.fori_loop(0, CH, zrow_loop, 0)
    for k in range(STRIPE // CH):
      pltpu.sync_copy(rows, acc.at[pl.ds(sid * STRIPE + k * CH, CH), :])
    rem = STRIPE % CH
    if rem:
      pltpu.sync_copy(rows.at[pl.ds(0, rem), :],
                      acc.at[pl.ds(sid * STRIPE + (STRIPE // CH) * CH, rem), :])

    plsc.subcore_barrier()

    ebase = sid * (NCH * CH)

    def chunk(i, carry):
      off = ebase + i * CH
      pltpu.sync_copy(src.at[pl.ds(off, CH)], sbuf)
      pltpu.sync_copy(dst.at[pl.ds(off, CH)], dbuf)
      pltpu.sync_copy(typ.at[pl.ds(off, CH)], tbuf)

      def sub(j, c):
        sl = pl.ds(j * 16, 16)
        idxg[sl] = sbuf[sl] * 2 + cid
        idxs[sl] = tbuf[sl] * N + dbuf[sl]
        return c

      lax.fori_loop(0, CH // 16, sub, 0)
      pltpu.async_copy(h2.at[idxg], rows, sem).wait()
      pltpu.sync_copy(rows, acc.at[idxs], add=True)
      return carry

    lax.fori_loop(0, NCH, chunk, 0)
    plsc.subcore_barrier()

    pltpu.sync_copy(acc.at[pl.ds(sid * STRIPE, STRIPE), :],
                    sums.at[pl.ds(sid * STRIPE, STRIPE), pl.ds(cid * HD, HD)])

  return pl.kernel(body, out_type=out_type, mesh=mesh,
                   scratch_types=scratch,
                   compiler_params=pltpu.CompilerParams(
                       use_tc_tiling_on_sc=False))


def _sc_cnt():
  mesh = plsc.VectorSubcoreMesh(core_axis_name="c", subcore_axis_name="s",
                                num_cores=2, num_subcores=NS)
  out_type = (jax.ShapeDtypeStruct((CNTN,), jnp.float32),)
  scratch = (
      pltpu.VMEM((CH,), jnp.int32),
      pltpu.VMEM((CH,), jnp.int32),
      pltpu.VMEM((CH,), jnp.int32),
      pltpu.VMEM((CH,), jnp.float32),
      pltpu.VMEM((CS + 16,), jnp.float32),
      pltpu.VMEM_SHARED((CNTN,), jnp.float32),
  )

  def body(dst, typ, cnt_out, dbuf, tbuf, idxs, ones_v, zcol, acccnt):
    cid = lax.axis_index("c")
    sid = lax.axis_index("s")
    zero16 = jnp.zeros((16,), jnp.float32)

    def ofill(j, c):
      ones_v[pl.ds(j * 16, 16)] = jnp.ones((16,), jnp.float32)
      return c

    lax.fori_loop(0, CH // 16, ofill, 0)

    def zfill(j, c):
      zcol[pl.ds(j * 16, 16)] = zero16
      return c

    lax.fori_loop(0, (CS + 16) // 16, zfill, 0)
    pltpu.sync_copy(zcol.at[pl.ds(0, CS)], acccnt.at[pl.ds(sid * CS, CS)])

    @pl.when(sid == 0)
    def _():
      pltpu.sync_copy(zcol.at[pl.ds(0, CNTN - NS * CS)],
                      acccnt.at[pl.ds(NS * CS, CNTN - NS * CS)])

    plsc.subcore_barrier()

    @pl.when(cid == 0)
    def _():
      ebase = sid * (NCH * CH)

      def chunk(i, carry):
        off = ebase + i * CH
        pltpu.sync_copy(dst.at[pl.ds(off, CH)], dbuf)
        pltpu.sync_copy(typ.at[pl.ds(off, CH)], tbuf)

        def sub(j, c):
          sl = pl.ds(j * 16, 16)
          idxs[sl] = tbuf[sl] * N + dbuf[sl]
          return c

        lax.fori_loop(0, CH // 16, sub, 0)
        pltpu.sync_copy(ones_v, acccnt.at[idxs], add=True)
        return carry

      lax.fori_loop(0, NCH, chunk, 0)

    plsc.subcore_barrier()

    @pl.when(cid == 0)
    def _():
      pltpu.sync_copy(acccnt.at[pl.ds(sid * CS, CS)],
                      cnt_out.at[pl.ds(sid * CS, CS)])

    @pl.when(jnp.logical_and(cid == 0, sid == 0))
    def _():
      pltpu.sync_copy(acccnt.at[pl.ds(NS * CS, CNTN - NS * CS)],
                      cnt_out.at[pl.ds(NS * CS, CNTN - NS * CS)])

  return pl.kernel(body, out_type=out_type, mesh=mesh,
                   scratch_types=scratch,
                   compiler_params=pltpu.CompilerParams(
                       use_tc_tiling_on_sc=False))


def _ln(h, g, b, eps=1e-5):
  mu = jnp.mean(h, axis=-1, keepdims=True)
  d = h - mu
  var = jnp.mean(d * d, axis=-1, keepdims=True)
  return d * lax.rsqrt(var + eps) * g + b


BR = 400


def _tc_layer(leaky, double_ln):
  def body(*args):
    if double_ln:
      (x_ref, s0_ref, s1_ref, s2_ref, cnt_ref, w_ref, root_ref, b_ref,
       g_ref, be_ref, g2_ref, be2_ref, o_ref) = args
    else:
      (x_ref, s0_ref, s1_ref, s2_ref, cnt_ref, w_ref, root_ref, b_ref,
       g_ref, be_ref, o_ref) = args

    x = x_ref[...]
    acc = jnp.dot(x, root_ref[...], precision=lax.Precision.HIGHEST,
                  preferred_element_type=jnp.float32) + b_ref[...][None, :]
    inv = 1.0 / jnp.maximum(cnt_ref[...], 1.0)
    for r, s_ref in enumerate((s0_ref, s1_ref, s2_ref)):
      mean_r = s_ref[...] * inv[:, r][:, None]
      acc = acc + jnp.dot(mean_r, w_ref[r], precision=lax.Precision.HIGHEST,
                          preferred_element_type=jnp.float32)
    h = _ln(acc, g_ref[...], be_ref[...])
    if leaky:
      h = jnp.where(h > 0, h, 0.2 * h)
    if double_ln:
      h = _ln(h, g2_ref[...], be2_ref[...])
    o_ref[...] = h

  grid = (N // BR,)
  nblk = N // BR
  vec = pl.BlockSpec((D,), lambda i: (0,))
  in_specs = [
      pl.BlockSpec((BR, D), lambda i: (i, 0)),
      pl.BlockSpec((BR, D), lambda i: (i, 0)),
      pl.BlockSpec((BR, D), lambda i: (i + nblk, 0)),
      pl.BlockSpec((BR, D), lambda i: (i + 2 * nblk, 0)),
      pl.BlockSpec((BR, R), lambda i: (i, 0)),
      pl.BlockSpec((R, D, D), lambda i: (0, 0, 0)),
      pl.BlockSpec((D, D), lambda i: (0, 0)),
      vec, vec, vec,
  ]
  if double_ln:
    in_specs += [vec, vec]
  return pl.pallas_call(
      body,
      grid=grid,
      in_specs=in_specs,
      out_specs=pl.BlockSpec((BR, D), lambda i: (i, 0)),
      out_shape=jax.ShapeDtypeStruct((N, D), jnp.float32),
  )


def kernel(x, edge_index, edge_type, w0, root0, b0, g0, be0,
           w1, root1, b1, g1, be1, gon, bon):
  E = edge_index.shape[1]
  pad = EP - E
  src = jnp.concatenate(
      [edge_index[0].astype(jnp.int32), jnp.zeros((pad,), jnp.int32)])
  dst = jnp.concatenate(
      [edge_index[1].astype(jnp.int32), jnp.zeros((pad,), jnp.int32)])
  typ = jnp.concatenate(
      [edge_type.astype(jnp.int32), jnp.full((pad,), R, jnp.int32)])

  agg = _sc_agg()
  cntk = _sc_cnt()
  tc0 = _tc_layer(leaky=True, double_ln=False)
  tc1 = _tc_layer(leaky=False, double_ln=True)

  (cnt,) = cntk(dst, typ)
  (sums0,) = agg(x.reshape(2 * N, HD), src, dst, typ)
  cnt_t = cnt[:R * N].reshape(R, N).T

  h1 = tc0(x, sums0, sums0, sums0, cnt_t, w0, root0, b0, g0, be0)

  (sums1,) = agg(h1.reshape(2 * N, HD), src, dst, typ)

  return tc1(h1, sums1, sums1, sums1, cnt_t, w1, root1, b1, g1, be1, gon, bon)
